# baseline (speedup 1.0000x reference)
import functools

import jax
import jax.numpy as jnp
from jax import lax
from jax.experimental import pallas as pl
from jax.experimental.pallas import tpu as pltpu
from jax.experimental.pallas import tpu_sc as plsc

BATCH = 1024
MEM_LEN = 200
POS_DIM = 128
MAX_SEQ_LEN = 48

ROWS = BATCH * MEM_LEN
NUM_WORKERS = 32
ROWS_PER_WORKER = ROWS // NUM_WORKERS
CHUNK = 200
NUM_CHUNKS = ROWS_PER_WORKER // CHUNK
NBUF = 4


def _sc_body(x_hbm, idx_hbm, tab_hbm, out_hbm, tab_sh,
             idx_bufs, x_bufs, sems_in, sems_add, sems_out):
    cid = lax.axis_index("c")
    sid = lax.axis_index("s")
    wid = sid * 2 + cid
    w_base = wid * ROWS_PER_WORKER

    def issue_in(ci, b):
        base = w_base + ci * CHUNK
        d1 = pltpu.async_copy(idx_hbm.at[pl.ds(base, CHUNK)], idx_bufs[b],
                              sems_in[b])
        d2 = pltpu.async_copy(x_hbm.at[pl.ds(base, CHUNK), :], x_bufs[b],
                              sems_in[b])
        return (d1, d2)

    def issue_add(b):
        return pltpu.async_copy(tab_sh.at[idx_bufs[b]], x_bufs[b],
                                sems_add[b], add=True)

    def issue_out(ci, b):
        base = w_base + ci * CHUNK
        return pltpu.async_copy(x_bufs[b], out_hbm.at[pl.ds(base, CHUNK), :],
                                sems_out[b])

    pend_in = [None] * NBUF
    pend_add = [None] * NBUF
    pend_out = [None] * NBUF

    pend_in[0] = issue_in(0, 0)
    if NUM_CHUNKS > 1:
        pend_in[1] = issue_in(1, 1)

    @pl.when(sid == 0)
    def _stage_table():
        pltpu.sync_copy(tab_hbm, tab_sh)

    plsc.subcore_barrier()

    for ci in range(NUM_CHUNKS):
        b = ci % NBUF
        for d in pend_in[b]:
            d.wait()
        pend_in[b] = None
        pend_add[b] = issue_add(b)
        if ci >= 1:
            pb = (ci - 1) % NBUF
            pend_add[pb].wait()
            pend_add[pb] = None
            pend_out[pb] = issue_out(ci - 1, pb)
        if ci + 2 < NUM_CHUNKS:
            nb = (ci + 2) % NBUF
            if pend_out[nb] is not None:
                pend_out[nb].wait()
                pend_out[nb] = None
            pend_in[nb] = issue_in(ci + 2, nb)

    lb = (NUM_CHUNKS - 1) % NBUF
    pend_add[lb].wait()
    pend_out[lb] = issue_out(NUM_CHUNKS - 1, lb)
    for b in range(NBUF):
        if pend_out[b] is not None:
            pend_out[b].wait()


@jax.jit
def _run(x_flat, idx_flat, table):
    mesh = plsc.VectorSubcoreMesh(core_axis_name="c", subcore_axis_name="s")

    def body(x_hbm, idx_hbm, tab_hbm, out_hbm, tab_sh, *rest):
        idx_bufs = rest[0:NBUF]
        x_bufs = rest[NBUF:2 * NBUF]
        sems_in = rest[2 * NBUF:3 * NBUF]
        sems_add = rest[3 * NBUF:4 * NBUF]
        sems_out = rest[4 * NBUF:5 * NBUF]
        _sc_body(x_hbm, idx_hbm, tab_hbm, out_hbm, tab_sh,
                 idx_bufs, x_bufs, sems_in, sems_add, sems_out)

    scratch = [pltpu.VMEM_SHARED((MAX_SEQ_LEN, POS_DIM), jnp.float32)]
    scratch += [pltpu.VMEM((CHUNK,), jnp.int32) for _ in range(NBUF)]
    scratch += [pltpu.VMEM((CHUNK, POS_DIM), jnp.float32) for _ in range(NBUF)]
    scratch += [pltpu.SemaphoreType.DMA for _ in range(3 * NBUF)]

    return pl.kernel(
        body,
        out_type=jax.ShapeDtypeStruct((ROWS, POS_DIM), jnp.float32),
        mesh=mesh,
        scratch_types=scratch,
    )(x_flat, idx_flat, table)


def kernel(x, pos_idx, embedding_weight):
    x_flat = x.reshape(ROWS, POS_DIM)
    idx_flat = pos_idx.reshape(ROWS).astype(jnp.int32)
    out = _run(x_flat, idx_flat, embedding_weight)
    return out.reshape(BATCH, MEM_LEN, POS_DIM)


# speedup vs baseline: 1.0116x; 1.0116x over previous
import functools

import jax
import jax.numpy as jnp
from jax import lax
from jax.experimental import pallas as pl
from jax.experimental.pallas import tpu as pltpu
from jax.experimental.pallas import tpu_sc as plsc

BATCH = 1024
MEM_LEN = 200
POS_DIM = 128
MAX_SEQ_LEN = 48

ROWS = BATCH * MEM_LEN
NUM_WORKERS = 32
ROWS_PER_WORKER = ROWS // NUM_WORKERS
CHUNK = 320
NUM_CHUNKS = ROWS_PER_WORKER // CHUNK
NBUF = 3


def _sc_body(x_hbm, idx_hbm, tab_hbm, out_hbm, tab_sh,
             idx_bufs, x_bufs, sems_in, sems_add, sems_out):
    cid = lax.axis_index("c")
    sid = lax.axis_index("s")
    wid = sid * 2 + cid
    w_base = wid * ROWS_PER_WORKER

    def issue_in(ci, b):
        base = w_base + ci * CHUNK
        d1 = pltpu.async_copy(idx_hbm.at[pl.ds(base, CHUNK)], idx_bufs[b],
                              sems_in[b])
        d2 = pltpu.async_copy(x_hbm.at[pl.ds(base, CHUNK), :], x_bufs[b],
                              sems_in[b])
        return (d1, d2)

    def issue_add(b):
        return pltpu.async_copy(tab_sh.at[idx_bufs[b]], x_bufs[b],
                                sems_add[b], add=True)

    def issue_out(ci, b):
        base = w_base + ci * CHUNK
        return pltpu.async_copy(x_bufs[b], out_hbm.at[pl.ds(base, CHUNK), :],
                                sems_out[b])

    pend_in = [None] * NBUF
    pend_add = [None] * NBUF
    pend_out = [None] * NBUF

    pend_in[0] = issue_in(0, 0)
    if NUM_CHUNKS > 1:
        pend_in[1] = issue_in(1, 1)

    @pl.when(sid == 0)
    def _stage_table():
        pltpu.sync_copy(tab_hbm, tab_sh)

    plsc.subcore_barrier()

    for ci in range(NUM_CHUNKS):
        b = ci % NBUF
        for d in pend_in[b]:
            d.wait()
        pend_in[b] = None
        pend_add[b] = issue_add(b)
        if ci >= 1:
            pb = (ci - 1) % NBUF
            pend_add[pb].wait()
            pend_add[pb] = None
            pend_out[pb] = issue_out(ci - 1, pb)
        if ci + 2 < NUM_CHUNKS:
            nb = (ci + 2) % NBUF
            if pend_out[nb] is not None:
                pend_out[nb].wait()
                pend_out[nb] = None
            pend_in[nb] = issue_in(ci + 2, nb)

    lb = (NUM_CHUNKS - 1) % NBUF
    pend_add[lb].wait()
    pend_out[lb] = issue_out(NUM_CHUNKS - 1, lb)
    for b in range(NBUF):
        if pend_out[b] is not None:
            pend_out[b].wait()


@jax.jit
def _run(x_flat, idx_flat, table):
    mesh = plsc.VectorSubcoreMesh(core_axis_name="c", subcore_axis_name="s")

    def body(x_hbm, idx_hbm, tab_hbm, out_hbm, tab_sh, *rest):
        idx_bufs = rest[0:NBUF]
        x_bufs = rest[NBUF:2 * NBUF]
        sems_in = rest[2 * NBUF:3 * NBUF]
        sems_add = rest[3 * NBUF:4 * NBUF]
        sems_out = rest[4 * NBUF:5 * NBUF]
        _sc_body(x_hbm, idx_hbm, tab_hbm, out_hbm, tab_sh,
                 idx_bufs, x_bufs, sems_in, sems_add, sems_out)

    scratch = [pltpu.VMEM_SHARED((MAX_SEQ_LEN, POS_DIM), jnp.float32)]
    scratch += [pltpu.VMEM((CHUNK,), jnp.int32) for _ in range(NBUF)]
    scratch += [pltpu.VMEM((CHUNK, POS_DIM), jnp.float32) for _ in range(NBUF)]
    scratch += [pltpu.SemaphoreType.DMA for _ in range(3 * NBUF)]

    return pl.kernel(
        body,
        out_type=jax.ShapeDtypeStruct((ROWS, POS_DIM), jnp.float32),
        mesh=mesh,
        scratch_types=scratch,
    )(x_flat, idx_flat, table)


def kernel(x, pos_idx, embedding_weight):
    x_flat = x.reshape(ROWS, POS_DIM)
    idx_flat = pos_idx.reshape(ROWS).astype(jnp.int32)
    out = _run(x_flat, idx_flat, embedding_weight)
    return out.reshape(BATCH, MEM_LEN, POS_DIM)
